# Initial kernel scaffold; baseline (speedup 1.0000x reference)
import jax
import jax.numpy as jnp
from jax.experimental import pallas as pl


def kernel(xyz):
    raise NotImplementedError("write your pallas kernel here")


# speedup vs baseline: 1.5051x; 1.5051x over previous
import functools

import jax
import jax.numpy as jnp
from jax.experimental import pallas as pl
from jax.experimental.pallas import tpu as pltpu

B = 16
N = 8192
G = 512
K = 32


def _fps_kernel(x_ref, y_ref, z_ref, idx_ref, cx_ref, cy_ref, cz_ref, dist_ref):
    lane = jax.lax.broadcasted_iota(jnp.int32, (B, N), 1)
    col = jax.lax.broadcasted_iota(jnp.int32, (B, G), 1)
    dist_ref[...] = jnp.full((B, N), 1e10, dtype=jnp.float32)
    x = x_ref[...]
    y = y_ref[...]
    z = z_ref[...]

    def body(g, carry):
        far, idxa, cxa, cya, cza = carry
        onehot = lane == far
        cx = jnp.sum(jnp.where(onehot, x, 0.0), axis=1, keepdims=True)
        cy = jnp.sum(jnp.where(onehot, y, 0.0), axis=1, keepdims=True)
        cz = jnp.sum(jnp.where(onehot, z, 0.0), axis=1, keepdims=True)
        emit = col == g
        idxa = jnp.where(emit, jnp.broadcast_to(far, (B, G)), idxa)
        cxa = jnp.where(emit, jnp.broadcast_to(cx, (B, G)), cxa)
        cya = jnp.where(emit, jnp.broadcast_to(cy, (B, G)), cya)
        cza = jnp.where(emit, jnp.broadcast_to(cz, (B, G)), cza)
        dx = x - cx
        dy = y - cy
        dz = z - cz
        d = (dx * dx + dy * dy) + dz * dz
        dist = jnp.minimum(dist_ref[...], d)
        dist_ref[...] = dist
        maxd = jnp.max(dist, axis=1, keepdims=True)
        new_far = jnp.min(jnp.where(dist == maxd, lane, N), axis=1,
                          keepdims=True).astype(jnp.int32)
        return new_far, idxa, cxa, cya, cza

    idx_ref[...] = jnp.zeros((B, G), jnp.int32)
    cx_ref[...] = jnp.zeros((B, G), jnp.float32)
    cy_ref[...] = jnp.zeros((B, G), jnp.float32)
    cz_ref[...] = jnp.zeros((B, G), jnp.float32)
    init = (jnp.zeros((B, 1), jnp.int32), idx_ref[...],
            cx_ref[...], cy_ref[...], cz_ref[...])
    _, idxa, cxa, cya, cza = jax.lax.fori_loop(0, G, body, init)
    idx_ref[...] = idxa
    cx_ref[...] = cxa
    cy_ref[...] = cya
    cz_ref[...] = cza


@functools.partial(jax.jit, static_argnums=())
def _fps(xs, ys, zs):
    out = pl.pallas_call(
        _fps_kernel,
        out_shape=(
            jax.ShapeDtypeStruct((B, G), jnp.int32),
            jax.ShapeDtypeStruct((B, G), jnp.float32),
            jax.ShapeDtypeStruct((B, G), jnp.float32),
            jax.ShapeDtypeStruct((B, G), jnp.float32),
        ),
        scratch_shapes=[pltpu.VMEM((B, N), jnp.float32)],
    )(xs, ys, zs)
    return out


def kernel(xyz):
    xs = xyz[:, :, 0]
    ys = xyz[:, :, 1]
    zs = xyz[:, :, 2]
    center_idx, cx, cy, cz = _fps(xs, ys, zs)
    center = jnp.stack([cx, cy, cz], axis=-1)

    d2 = (jnp.sum(center ** 2, axis=-1)[:, :, None]
          + jnp.sum(xyz ** 2, axis=-1)[:, None, :]
          - 2.0 * jnp.einsum('bgd,bnd->bgn', center, xyz))
    dists = jnp.sqrt(jnp.maximum(d2, 0.0))
    knn_idx = jax.lax.top_k(-jax.lax.stop_gradient(dists), K)[1]
    neighborhood = jax.vmap(lambda pts, idx: pts[idx])(xyz, knn_idx)
    neighborhood = neighborhood - center[:, :, None, :]
    return (neighborhood, center)


# speedup vs baseline: 5.0000x; 3.3220x over previous
import functools

import jax
import jax.numpy as jnp
from jax.experimental import pallas as pl
from jax.experimental.pallas import tpu as pltpu

B = 16
N = 8192
G = 512
K = 32
GT = 64
C = 64
W = 128


def _fps_kernel(x_ref, y_ref, z_ref, idx_ref, cx_ref, cy_ref, cz_ref,
                n2_ref, c2_ref, dist_ref):
    lane = jax.lax.broadcasted_iota(jnp.int32, (B, N), 1)
    col = jax.lax.broadcasted_iota(jnp.int32, (B, G), 1)
    dist_ref[...] = jnp.full((B, N), 1e10, dtype=jnp.float32)
    x = x_ref[...]
    y = y_ref[...]
    z = z_ref[...]
    n2_ref[...] = (x * x + z * z) + y * y

    def body(g, carry):
        far, idxa, cxa, cya, cza = carry
        onehot = lane == far
        cx = jnp.sum(jnp.where(onehot, x, 0.0), axis=1, keepdims=True)
        cy = jnp.sum(jnp.where(onehot, y, 0.0), axis=1, keepdims=True)
        cz = jnp.sum(jnp.where(onehot, z, 0.0), axis=1, keepdims=True)
        emit = col == g
        idxa = jnp.where(emit, jnp.broadcast_to(far, (B, G)), idxa)
        cxa = jnp.where(emit, jnp.broadcast_to(cx, (B, G)), cxa)
        cya = jnp.where(emit, jnp.broadcast_to(cy, (B, G)), cya)
        cza = jnp.where(emit, jnp.broadcast_to(cz, (B, G)), cza)
        dx = x - cx
        dy = y - cy
        dz = z - cz
        d = (dx * dx + dz * dz) + dy * dy
        dist = jnp.minimum(dist_ref[...], d)
        dist_ref[...] = dist
        maxd = jnp.max(dist, axis=1, keepdims=True)
        new_far = jnp.min(jnp.where(dist == maxd, lane, N), axis=1,
                          keepdims=True).astype(jnp.int32)
        return new_far, idxa, cxa, cya, cza

    idx_ref[...] = jnp.zeros((B, G), jnp.int32)
    cx_ref[...] = jnp.zeros((B, G), jnp.float32)
    cy_ref[...] = jnp.zeros((B, G), jnp.float32)
    cz_ref[...] = jnp.zeros((B, G), jnp.float32)
    init = (jnp.zeros((B, 1), jnp.int32), idx_ref[...],
            cx_ref[...], cy_ref[...], cz_ref[...])
    _, idxa, cxa, cya, cza = jax.lax.fori_loop(0, G, body, init)
    idx_ref[...] = idxa
    cx_ref[...] = cxa
    cy_ref[...] = cya
    cz_ref[...] = cza
    c2_ref[...] = (cxa * cxa + cza * cza) + cya * cya


def _fps(xs, ys, zs):
    return pl.pallas_call(
        _fps_kernel,
        out_shape=(
            jax.ShapeDtypeStruct((B, G), jnp.int32),
            jax.ShapeDtypeStruct((B, G), jnp.float32),
            jax.ShapeDtypeStruct((B, G), jnp.float32),
            jax.ShapeDtypeStruct((B, G), jnp.float32),
            jax.ShapeDtypeStruct((B, N), jnp.float32),
            jax.ShapeDtypeStruct((B, G), jnp.float32),
        ),
        scratch_shapes=[pltpu.VMEM((B, N), jnp.float32)],
    )(xs, ys, zs)


def _ce(av, ai, bv, bi):
    m = (av < bv) | ((av == bv) & (ai <= bi))
    lv = jnp.where(m, av, bv)
    li = jnp.where(m, ai, bi)
    hv = jnp.where(m, bv, av)
    hi = jnp.where(m, bi, ai)
    return (lv, li), (hv, hi)


def _bitonic_merge(vals, idxs, lo, n, ascending):
    if n <= 1:
        return
    half = n // 2
    for i in range(lo, lo + half):
        j = i + half
        (lv, li), (hv, hi) = _ce(vals[i], idxs[i], vals[j], idxs[j])
        if ascending:
            vals[i], idxs[i], vals[j], idxs[j] = lv, li, hv, hi
        else:
            vals[i], idxs[i], vals[j], idxs[j] = hv, hi, lv, li
    _bitonic_merge(vals, idxs, lo, half, ascending)
    _bitonic_merge(vals, idxs, lo + half, half, ascending)


def _bitonic_sort(vals, idxs, lo, n, ascending):
    if n <= 1:
        return
    half = n // 2
    _bitonic_sort(vals, idxs, lo, half, True)
    _bitonic_sort(vals, idxs, lo + half, half, False)
    _bitonic_merge(vals, idxs, lo, n, ascending)


def _knn_kernel(e_ref, n2_ref, c2_ref, out_ref):
    dot = e_ref[0]
    n2 = n2_ref[0]
    c2 = c2_ref[0]
    c2col = jnp.broadcast_to(c2, (GT, N))
    d2 = (c2col + jnp.broadcast_to(n2, (GT, N))) - 2.0 * dot
    d2 = jnp.sqrt(jnp.maximum(d2, 0.0))

    lane128 = jax.lax.broadcasted_iota(jnp.int32, (GT, W), 1)
    vals = []
    idxs = []
    for c in range(C):
        vals.append(jax.lax.slice(d2, (0, c * W), (GT, (c + 1) * W)))
        idxs.append(lane128 + (c * W))

    _bitonic_sort(vals, idxs, 0, C, True)

    vals = vals[:K]
    idxs = idxs[:K]

    INF = jnp.float32(3e38)
    BIGI = jnp.int32(2 ** 30)
    outs = []
    for s in range(K):
        top = vals[0]
        minv = jnp.min(top, axis=1, keepdims=True)
        sel_idx = jnp.min(jnp.where(top == minv, idxs[0], BIGI), axis=1,
                          keepdims=True)
        selmask = idxs[0] == sel_idx
        outs.append(sel_idx)
        dl = K - s - 1
        for j in range(dl):
            vals[j] = jnp.where(selmask, vals[j + 1], vals[j])
            idxs[j] = jnp.where(selmask, idxs[j + 1], idxs[j])
        vals[dl] = jnp.where(selmask, INF, vals[dl])
        idxs[dl] = jnp.where(selmask, BIGI, idxs[dl])

    out_ref[0] = jnp.concatenate(outs, axis=1)


def _knn(e, n2, c2):
    grid = (B, G // GT)
    return pl.pallas_call(
        _knn_kernel,
        grid=grid,
        in_specs=[
            pl.BlockSpec((1, GT, N), lambda b, t: (b, t, 0)),
            pl.BlockSpec((1, 1, N), lambda b, t: (b, 0, 0)),
            pl.BlockSpec((1, GT, 1), lambda b, t: (b, t, 0)),
        ],
        out_specs=pl.BlockSpec((1, GT, K), lambda b, t: (b, t, 0)),
        out_shape=jax.ShapeDtypeStruct((B, G, K), jnp.int32),
    )(e, n2.reshape(B, 1, N), c2.reshape(B, G, 1))


def kernel(xyz):
    xs = xyz[:, :, 0]
    ys = xyz[:, :, 1]
    zs = xyz[:, :, 2]
    center_idx, cx, cy, cz, n2k, c2k = _fps(xs, ys, zs)
    center = jnp.stack([cx, cy, cz], axis=-1)
    n2 = jnp.sum(xyz ** 2, axis=-1)
    c2 = jnp.sum(center ** 2, axis=-1)
    e = jnp.einsum('bgd,bnd->bgn', center, xyz)
    knn_idx = _knn(e, n2, c2)
    neighborhood = jax.vmap(lambda pts, idx: pts[idx])(xyz, knn_idx)
    neighborhood = neighborhood - center[:, :, None, :]
    return (neighborhood, center)
